# baseline (speedup 1.0000x reference)
import jax
import jax.numpy as jnp
from jax.experimental import pallas as pl
from jax.experimental.pallas import tpu as pltpu

BATCH = 64
NUM_PATCHES = 1024
PROJ_DIM = 768

CB = 8
NCHUNK = BATCH // CB
NBUF = 2
REFILL_LAG = 1


def _add_body(x_hbm, pos_hbm, o_hbm, pos_v, *rest):
    bufs = rest[:NBUF]
    lds = rest[NBUF : 2 * NBUF]
    sts = rest[2 * NBUF : 3 * NBUF]
    psem = rest[3 * NBUF]

    pltpu.make_async_copy(pos_hbm, pos_v, psem).start()

    for k in range(NBUF):
        pltpu.make_async_copy(x_hbm.at[pl.ds(k * CB, CB)], bufs[k], lds[k]).start()

    pltpu.make_async_copy(pos_hbm, pos_v, psem).wait()

    for c in range(NCHUNK):
        k = c % NBUF
        sl = pl.ds(c * CB, CB)
        pltpu.make_async_copy(x_hbm.at[sl], bufs[k], lds[k]).wait()
        bufs[k][...] = bufs[k][...] + pos_v[...][None]
        pltpu.make_async_copy(bufs[k], o_hbm.at[sl], sts[k]).start()

        cr = c + REFILL_LAG
        if NBUF <= cr < NCHUNK:
            kr = cr % NBUF
            prev = pl.ds((cr - NBUF) * CB, CB)
            pltpu.make_async_copy(bufs[kr], o_hbm.at[prev], sts[kr]).wait()
            pltpu.make_async_copy(x_hbm.at[pl.ds(cr * CB, CB)], bufs[kr], lds[kr]).start()

    for c in range(NCHUNK - NBUF, NCHUNK):
        k = c % NBUF
        pltpu.make_async_copy(bufs[k], o_hbm.at[pl.ds(c * CB, CB)], sts[k]).wait()


def kernel(encoded_patches, pos_table):
    B, P, D = encoded_patches.shape
    return pl.pallas_call(
        _add_body,
        in_specs=[
            pl.BlockSpec(memory_space=pltpu.MemorySpace.HBM),
            pl.BlockSpec(memory_space=pltpu.MemorySpace.HBM),
        ],
        out_specs=pl.BlockSpec(memory_space=pltpu.MemorySpace.HBM),
        out_shape=jax.ShapeDtypeStruct((B, P, D), encoded_patches.dtype),
        scratch_shapes=(
            [pltpu.VMEM((P, D), jnp.float32)]
            + [pltpu.VMEM((CB, P, D), jnp.float32) for _ in range(NBUF)]
            + [pltpu.SemaphoreType.DMA for _ in range(2 * NBUF + 1)]
        ),
        compiler_params=pltpu.CompilerParams(vmem_limit_bytes=64 * 1024 * 1024),
    )(encoded_patches, pos_table)


# speedup vs baseline: 1.1816x; 1.1816x over previous
import jax
import jax.numpy as jnp
from jax.experimental import pallas as pl
from jax.experimental.pallas import tpu as pltpu


def _add_kernel(x_ref, pos_ref, o_ref):
    o_ref[...] = x_ref[...] + pos_ref[...]


def kernel(encoded_patches, pos_table):
    B, P, D = encoded_patches.shape
    BB = 5
    grid = ((B + BB - 1) // BB,)
    return pl.pallas_call(
        _add_kernel,
        grid=grid,
        in_specs=[
            pl.BlockSpec((BB, P, D), lambda b: (b, 0, 0)),
            pl.BlockSpec((P, D), lambda b: (0, 0)),
        ],
        out_specs=pl.BlockSpec((BB, P, D), lambda b: (b, 0, 0)),
        out_shape=jax.ShapeDtypeStruct((B, P, D), encoded_patches.dtype),
        compiler_params=pltpu.CompilerParams(vmem_limit_bytes=128 * 1024 * 1024),
    )(encoded_patches, pos_table)
